# baseline (speedup 1.0000x reference)
import dataclasses
import functools

import jax
import jax.numpy as jnp
from jax import lax
from jax.experimental import pallas as pl
from jax.experimental.pallas import tpu as pltpu
from jax.experimental.pallas import tpu_sc as plsc

ROWS = 128
COLS = 8192
LANES = 16
NUM_CORES = 2
NUM_SUBCORES = 16
NUM_WORKERS = NUM_CORES * NUM_SUBCORES
ROWS_PER_W = ROWS // NUM_WORKERS
NUM_VECS = COLS // LANES


def _scan_kernel(x_hbm, o_hbm, buf, sem):
    wid = lax.axis_index("c") * NUM_SUBCORES + lax.axis_index("s")
    base = wid * ROWS_PER_W
    pltpu.async_copy(x_hbm.at[pl.ds(base, ROWS_PER_W)], buf, sem).wait()

    def body(i, carries):
        new = []
        for r in range(ROWS_PER_W):
            v = buf[r, pl.ds(i * LANES, LANES)]
            buf[r, pl.ds(i * LANES, LANES)] = plsc.cumsum(v) + carries[r]
            new.append(carries[r] + jnp.sum(v))
        return tuple(new)

    lax.fori_loop(0, NUM_VECS, body,
                  tuple(jnp.float32(0.0) for _ in range(ROWS_PER_W)))

    pltpu.async_copy(buf, o_hbm.at[pl.ds(base, ROWS_PER_W)], sem).wait()


def kernel(x):
    mesh = plsc.VectorSubcoreMesh(core_axis_name="c", subcore_axis_name="s")
    cp = pltpu.CompilerParams()
    if "needs_layout_passes" in pltpu.CompilerParams.__dataclass_fields__:
        cp = dataclasses.replace(cp, needs_layout_passes=False)
    run = functools.partial(
        pl.kernel,
        out_type=jax.ShapeDtypeStruct((ROWS, COLS), jnp.float32),
        mesh=mesh,
        compiler_params=cp,
        scratch_types=[
            pltpu.VMEM((ROWS_PER_W, COLS), jnp.float32),
            pltpu.SemaphoreType.DMA,
        ],
    )(_scan_kernel)
    return run(x)
